# baseline (speedup 1.0000x reference)
import functools

import jax
import jax.numpy as jnp
from jax import lax
from jax.experimental import pallas as pl
from jax.experimental.pallas import tpu as pltpu
from jax.experimental.pallas import tpu_sc as plsc

NUM_E = 1000000
NUM_R = 1000
ED = 64
RD = 32
B = 16384
PF = ED * RD

NC = 2
NS = 16
NW = NC * NS
BPW = B // NW
ECH = 128
PCH = 32


def _sc_gather_body(heads_hbm, rels_hbm, tails_hbm, ent_hbm, remb_hbm, proj_hbm,
                    hg_hbm, tg_hbm, rg_hbm, pg_hbm,
                    idx_h, idx_t, idx_r, ebuf, rbuf, pbuf, sem):
    wid = lax.axis_index("s") * NC + lax.axis_index("c")
    base = wid * BPW
    pltpu.sync_copy(heads_hbm.at[pl.ds(base, BPW)], idx_h)
    pltpu.sync_copy(tails_hbm.at[pl.ds(base, BPW)], idx_t)
    pltpu.sync_copy(rels_hbm.at[pl.ds(base, BPW)], idx_r)

    @pl.loop(0, BPW, step=ECH)
    def _h(c):
        pltpu.async_copy(ent_hbm.at[idx_h.at[pl.ds(c, ECH)]], ebuf, sem).wait()
        pltpu.sync_copy(ebuf, hg_hbm.at[pl.ds(base + c, ECH)])

    @pl.loop(0, BPW, step=ECH)
    def _t(c):
        pltpu.async_copy(ent_hbm.at[idx_t.at[pl.ds(c, ECH)]], ebuf, sem).wait()
        pltpu.sync_copy(ebuf, tg_hbm.at[pl.ds(base + c, ECH)])

    @pl.loop(0, BPW, step=ECH)
    def _r(c):
        pltpu.async_copy(remb_hbm.at[idx_r.at[pl.ds(c, ECH)]], rbuf, sem).wait()
        pltpu.sync_copy(rbuf, rg_hbm.at[pl.ds(base + c, ECH)])

    @pl.loop(0, BPW, step=PCH)
    def _p(c):
        pltpu.async_copy(proj_hbm.at[idx_r.at[pl.ds(c, PCH)]], pbuf, sem).wait()
        pltpu.sync_copy(pbuf, pg_hbm.at[pl.ds(base + c, PCH)])


def _sc_gather(heads, relations, tails, ent, remb, projf):
    f32 = jnp.float32
    return pl.kernel(
        _sc_gather_body,
        out_type=(
            jax.ShapeDtypeStruct((B, ED), f32),
            jax.ShapeDtypeStruct((B, ED), f32),
            jax.ShapeDtypeStruct((B, RD), f32),
            jax.ShapeDtypeStruct((B, PF), f32),
        ),
        mesh=plsc.VectorSubcoreMesh(core_axis_name="c", subcore_axis_name="s"),
        scratch_types=[
            pltpu.VMEM((BPW,), jnp.int32),
            pltpu.VMEM((BPW,), jnp.int32),
            pltpu.VMEM((BPW,), jnp.int32),
            pltpu.VMEM((ECH, ED), f32),
            pltpu.VMEM((ECH, RD), f32),
            pltpu.VMEM((PCH, PF), f32),
            pltpu.SemaphoreType.DMA,
        ],
        compiler_params=pltpu.CompilerParams(use_tc_tiling_on_sc=False),
    )(heads, relations, tails, ent, remb, projf)


TT = 256


def _tc_score_body(h_ref, t_ref, r_ref, p_ref, o_ref):
    u = h_ref[...] - t_ref[...]
    pj = p_ref[...].reshape(TT, ED, RD)
    diff = jnp.sum(u[:, :, None] * pj, axis=1) + r_ref[...]
    o_ref[...] = jnp.sqrt(jnp.sum(diff * diff, axis=1))


def _tc_score(hg, tg, rg, pg):
    return pl.pallas_call(
        _tc_score_body,
        grid=(B // TT,),
        in_specs=[
            pl.BlockSpec((TT, ED), lambda i: (i, 0)),
            pl.BlockSpec((TT, ED), lambda i: (i, 0)),
            pl.BlockSpec((TT, RD), lambda i: (i, 0)),
            pl.BlockSpec((TT, PF), lambda i: (i, 0)),
        ],
        out_specs=pl.BlockSpec((TT,), lambda i: (i,)),
        out_shape=jax.ShapeDtypeStruct((B,), jnp.float32),
    )(hg, tg, rg, pg)


def kernel(heads, relations, tails, entity_embeddings, relation_embeddings,
           projection_matrices):
    projf = projection_matrices.reshape(NUM_R, PF)
    hg, tg, rg, pg = _sc_gather(heads, relations, tails, entity_embeddings,
                                relation_embeddings, projf)
    return _tc_score(hg, tg, rg, pg)


# speedup vs baseline: 1.4299x; 1.4299x over previous
import jax
import jax.numpy as jnp
from jax import lax
from jax.experimental import pallas as pl
from jax.experimental.pallas import tpu as pltpu
from jax.experimental.pallas import tpu_sc as plsc

NUM_E = 1000000
NUM_R = 1000
ED = 64
RD = 32
B = 16384
PF = ED * RD

NC = 2
NS = 16
NW = NC * NS
BPW = B // NW
ECH = 128
PCH = 32
LANES = 16


def _sc_gather_body(heads_hbm, rels_hbm, tails_hbm, ent2_hbm, remb2_hbm,
                    proj_hbm, hg_hbm, tg_hbm, rg_hbm, pg_hbm,
                    idx_h, idx_t, idx_r, idx_r2, ebuf, pbuf, sem):
    wid = lax.axis_index("s") * NC + lax.axis_index("c")
    base = wid * BPW
    pltpu.sync_copy(heads_hbm.at[pl.ds(base, BPW)], idx_h)
    pltpu.sync_copy(tails_hbm.at[pl.ds(base, BPW)], idx_t)
    pltpu.sync_copy(rels_hbm.at[pl.ds(base, BPW)], idx_r)

    @pl.loop(0, BPW, step=LANES)
    def _shift(i):
        slc = pl.ds(i, LANES)
        idx_h[slc] = lax.shift_right_logical(idx_h[slc], 1)
        idx_t[slc] = lax.shift_right_logical(idx_t[slc], 1)
        idx_r2[slc] = lax.shift_right_logical(idx_r[slc], 2)

    @pl.loop(0, BPW, step=ECH)
    def _h(c):
        pltpu.async_copy(ent2_hbm.at[idx_h.at[pl.ds(c, ECH)]], ebuf, sem).wait()
        pltpu.sync_copy(ebuf, hg_hbm.at[pl.ds(base + c, ECH)])

    @pl.loop(0, BPW, step=ECH)
    def _t(c):
        pltpu.async_copy(ent2_hbm.at[idx_t.at[pl.ds(c, ECH)]], ebuf, sem).wait()
        pltpu.sync_copy(ebuf, tg_hbm.at[pl.ds(base + c, ECH)])

    @pl.loop(0, BPW, step=ECH)
    def _r(c):
        pltpu.async_copy(remb2_hbm.at[idx_r2.at[pl.ds(c, ECH)]], ebuf, sem).wait()
        pltpu.sync_copy(ebuf, rg_hbm.at[pl.ds(base + c, ECH)])

    @pl.loop(0, BPW, step=PCH)
    def _p(c):
        pltpu.async_copy(proj_hbm.at[idx_r.at[pl.ds(c, PCH)]], pbuf, sem).wait()
        pltpu.sync_copy(pbuf, pg_hbm.at[pl.ds(base + c, PCH)])


def _sc_gather(heads, relations, tails, ent2, remb2, projf):
    f32 = jnp.float32
    return pl.kernel(
        _sc_gather_body,
        out_type=(
            jax.ShapeDtypeStruct((B, 128), f32),
            jax.ShapeDtypeStruct((B, 128), f32),
            jax.ShapeDtypeStruct((B, 128), f32),
            jax.ShapeDtypeStruct((B, PF), f32),
        ),
        mesh=plsc.VectorSubcoreMesh(core_axis_name="c", subcore_axis_name="s"),
        scratch_types=[
            pltpu.VMEM((BPW,), jnp.int32),
            pltpu.VMEM((BPW,), jnp.int32),
            pltpu.VMEM((BPW,), jnp.int32),
            pltpu.VMEM((BPW,), jnp.int32),
            pltpu.VMEM((ECH, 128), f32),
            pltpu.VMEM((PCH, PF), f32),
            pltpu.SemaphoreType.DMA,
        ],
    )(heads, relations, tails, ent2, remb2, projf)


TT = 256


def _tc_score_body(h_idx_ref, r_idx_ref, t_idx_ref, hg_ref, tg_ref, rg_ref,
                   pg_ref, rsel_ref, o_ref):
    hh = hg_ref[...]
    tt = tg_ref[...]
    hpar = (h_idx_ref[...] & 1)[:, None] == 1
    tpar = (t_idx_ref[...] & 1)[:, None] == 1
    h = jnp.where(hpar, hh[:, ED:], hh[:, :ED])
    t = jnp.where(tpar, tt[:, ED:], tt[:, :ED])
    u = h - t
    ue = jax.lax.dot(u, rsel_ref[...],
                     preferred_element_type=jnp.float32)
    prod = ue * pg_ref[...]
    s = prod[:, 0:128]
    for c in range(1, PF // 128):
        s = s + prod[:, 128 * c:128 * (c + 1)]
    s4 = s[:, 0:32] + s[:, 32:64] + s[:, 64:96] + s[:, 96:128]
    rm = (r_idx_ref[...] & 3)[:, None]
    rr = rg_ref[...]
    rq = jnp.where(rm == 0, rr[:, 0:32],
                   jnp.where(rm == 1, rr[:, 32:64],
                             jnp.where(rm == 2, rr[:, 64:96], rr[:, 96:128])))
    diff = s4 + rq
    o_ref[...] = jnp.sqrt(jnp.sum(diff * diff, axis=1))


def _tc_score(heads, relations, tails, hg, tg, rg, pg, rsel):
    return pl.pallas_call(
        _tc_score_body,
        grid=(B // TT,),
        in_specs=[
            pl.BlockSpec((TT,), lambda i: (i,)),
            pl.BlockSpec((TT,), lambda i: (i,)),
            pl.BlockSpec((TT,), lambda i: (i,)),
            pl.BlockSpec((TT, 128), lambda i: (i, 0)),
            pl.BlockSpec((TT, 128), lambda i: (i, 0)),
            pl.BlockSpec((TT, 128), lambda i: (i, 0)),
            pl.BlockSpec((TT, PF), lambda i: (i, 0)),
            pl.BlockSpec((ED, PF), lambda i: (0, 0)),
        ],
        out_specs=pl.BlockSpec((TT,), lambda i: (i,)),
        out_shape=jax.ShapeDtypeStruct((B,), jnp.float32),
    )(heads, relations, tails, hg, tg, rg, pg, rsel)


def kernel(heads, relations, tails, entity_embeddings, relation_embeddings,
           projection_matrices):
    ent2 = entity_embeddings.reshape(NUM_E // 2, 128)
    remb2 = relation_embeddings.reshape(NUM_R // 4, 128)
    projf = projection_matrices.reshape(NUM_R, PF)
    k = jnp.arange(PF, dtype=jnp.int32)[None, :]
    d = jnp.arange(ED, dtype=jnp.int32)[:, None]
    rsel = (k // RD == d).astype(jnp.float32)
    hg, tg, rg, pg = _sc_gather(heads, relations, tails, ent2, remb2, projf)
    return _tc_score(heads, relations, tails, hg, tg, rg, pg, rsel)


# speedup vs baseline: 1.5239x; 1.0657x over previous
import jax
import jax.numpy as jnp
from jax import lax
from jax.experimental import pallas as pl
from jax.experimental.pallas import tpu as pltpu
from jax.experimental.pallas import tpu_sc as plsc

NUM_E = 1000000
NUM_R = 1000
ED = 64
RD = 32
B = 16384
PF = ED * RD
PW = 2176

NC = 2
NS = 16
NW = NC * NS
BPW = B // NW
ECH = 128
PCH = 32


def _sc_ent_body(heads_hbm, tails_hbm, ent_hbm, hb_hbm,
                 idx_h, idx_t, gbuf, cbuf, sem):
    wid = lax.axis_index("s") * NC + lax.axis_index("c")
    base = wid * BPW
    pltpu.sync_copy(heads_hbm.at[pl.ds(base, BPW)], idx_h)
    pltpu.sync_copy(tails_hbm.at[pl.ds(base, BPW)], idx_t)

    @pl.loop(0, BPW, step=ECH)
    def _h(c):
        pltpu.async_copy(ent_hbm.at[idx_h.at[pl.ds(c, ECH)]], gbuf, sem).wait()
        pltpu.sync_copy(gbuf, hb_hbm.at[pl.ds(base + c, ECH), pl.ds(0, ED)])
        pltpu.async_copy(ent_hbm.at[idx_t.at[pl.ds(c, ECH)]], cbuf, sem).wait()
        pltpu.sync_copy(cbuf, hb_hbm.at[pl.ds(base + c, ECH), pl.ds(ED, ED)])


def _sc_ent_gather(heads, tails, ent):
    f32 = jnp.float32
    return pl.kernel(
        _sc_ent_body,
        out_type=jax.ShapeDtypeStruct((B, 128), f32),
        mesh=plsc.VectorSubcoreMesh(core_axis_name="c", subcore_axis_name="s"),
        scratch_types=[
            pltpu.VMEM((BPW,), jnp.int32),
            pltpu.VMEM((BPW,), jnp.int32),
            pltpu.VMEM((ECH, ED), f32),
            pltpu.VMEM((ECH, ED), f32),
            pltpu.SemaphoreType.DMA,
        ],
        compiler_params=pltpu.CompilerParams(use_tc_tiling_on_sc=False),
    )(heads, tails, ent)


def _sc_proj_body(rels_hbm, proj_hbm, pg_hbm, idx_r, pbuf, sem):
    wid = lax.axis_index("s") * NC + lax.axis_index("c")
    base = wid * BPW
    pltpu.sync_copy(rels_hbm.at[pl.ds(base, BPW)], idx_r)

    @pl.loop(0, BPW, step=PCH)
    def _p(c):
        pltpu.async_copy(proj_hbm.at[idx_r.at[pl.ds(c, PCH)]], pbuf, sem).wait()
        pltpu.sync_copy(pbuf, pg_hbm.at[pl.ds(base + c, PCH)])


def _sc_proj_gather(relations, projaug):
    f32 = jnp.float32
    return pl.kernel(
        _sc_proj_body,
        out_type=jax.ShapeDtypeStruct((B, PW), f32),
        mesh=plsc.VectorSubcoreMesh(core_axis_name="c", subcore_axis_name="s"),
        scratch_types=[
            pltpu.VMEM((BPW,), jnp.int32),
            pltpu.VMEM((PCH, PW), f32),
            pltpu.SemaphoreType.DMA,
        ],
    )(relations, projaug)


TT = 256


def _tc_score_body(hb_ref, pg_ref, rsel_ref, o_ref):
    hb = hb_ref[...]
    u = hb[:, :ED] - hb[:, ED:]
    ue = jax.lax.dot(u, rsel_ref[...],
                     preferred_element_type=jnp.float32)
    prod = ue * pg_ref[:, :PF]
    s = prod[:, 0:128]
    for c in range(1, PF // 128):
        s = s + prod[:, 128 * c:128 * (c + 1)]
    s4 = s[:, 0:32] + s[:, 32:64] + s[:, 64:96] + s[:, 96:128]
    diff = s4 + pg_ref[:, PF:PF + RD]
    o_ref[...] = jnp.sqrt(jnp.sum(diff * diff, axis=1))


def _tc_score(hb, pg, rsel):
    return pl.pallas_call(
        _tc_score_body,
        grid=(B // TT,),
        in_specs=[
            pl.BlockSpec((TT, 128), lambda i: (i, 0)),
            pl.BlockSpec((TT, PW), lambda i: (i, 0)),
            pl.BlockSpec((ED, PF), lambda i: (0, 0)),
        ],
        out_specs=pl.BlockSpec((TT,), lambda i: (i,)),
        out_shape=jax.ShapeDtypeStruct((B,), jnp.float32),
    )(hb, pg, rsel)


def kernel(heads, relations, tails, entity_embeddings, relation_embeddings,
           projection_matrices):
    projaug = jnp.concatenate(
        [projection_matrices.reshape(NUM_R, PF), relation_embeddings,
         jnp.zeros((NUM_R, PW - PF - RD), jnp.float32)], axis=1)
    k = jnp.arange(PF, dtype=jnp.int32)[None, :]
    d = jnp.arange(ED, dtype=jnp.int32)[:, None]
    rsel = (k // RD == d).astype(jnp.float32)
    hb = _sc_ent_gather(heads, tails, entity_embeddings)
    pg = _sc_proj_gather(relations, projaug)
    return _tc_score(hb, pg, rsel)
